# baseline (speedup 1.0000x reference)
import functools

import jax
import jax.numpy as jnp
from jax import lax
from jax.experimental import pallas as pl
from jax.experimental.pallas import tpu as pltpu
from jax.experimental.pallas import tpu_sc as plsc

_N = 10000
_E = 160000
_D = 256
_NT = 3
_T = 4
_G = 64

_NC = 2
_NS = 16

_HALFN = 5120
_SLAB = 5248
_ROWS_PT = _SLAB // _NS
_DUMMY = 5200
_CHUNK = 64
_NBUF = 2
_SECT = 24
_CPS = _NS * _SECT
_MAXCH = 2688
_CAPR = 2 * _MAXCH + 32

_BLK = 1000
_NBLK = _N // _BLK


def _sc_agg_body(tr_hbm, gidx_hbm, didx_hbm, scal_hbm, zeros_hbm, out_hbm,
                 gidx_v, didx_v, scal0_v, buf0, buf1,
                 slab, sem0, sem1):
    c = lax.axis_index("c")
    s = lax.axis_index("s")

    pltpu.sync_copy(zeros_hbm, slab.at[pl.ds(s * _ROWS_PT, _ROWS_PT)])

    pltpu.sync_copy(scal_hbm.at[c], scal0_v)
    v = scal0_v[...]
    q0d8 = v[0] + s * v[1]
    nsec = v[2]
    plsc.subcore_barrier()

    bufs = (buf0, buf1)
    sems = (sem0, sem1)

    def section_body(sec, carry):
        base = (q0d8 + sec * (_SECT // 8)) * 8
        pltpu.sync_copy(gidx_hbm.at[pl.ds(base, _SECT)], gidx_v)
        pltpu.sync_copy(didx_hbm.at[pl.ds(base, _SECT)], didx_v)
        for b in range(_NBUF):
            pltpu.async_copy(tr_hbm.at[gidx_v.at[b]], bufs[b], sems[b])

        def ring_body(i, carry2):
            for b in range(_NBUF):
                j = i * _NBUF + b
                pltpu.make_async_copy(tr_hbm.at[gidx_v.at[j]],
                                      bufs[b], sems[b]).wait()
                pltpu.sync_copy(bufs[b], slab.at[didx_v.at[j]], add=True)

                @pl.when(j + _NBUF < _SECT)
                def _():
                    pltpu.async_copy(tr_hbm.at[gidx_v.at[j + _NBUF]],
                                     bufs[b], sems[b])
            return carry2

        lax.fori_loop(0, _SECT // _NBUF, ring_body, 0)
        return carry

    lax.fori_loop(0, nsec, section_body, 0)

    plsc.subcore_barrier()
    pltpu.sync_copy(slab.at[pl.ds(s * _ROWS_PT, _ROWS_PT)],
                    out_hbm.at[c, pl.ds(s * _ROWS_PT, _ROWS_PT)])


@functools.cache
def _get_sc_agg():
    return pl.kernel(
        _sc_agg_body,
        out_type=jax.ShapeDtypeStruct((_NC, _SLAB, 2, _D // 2), jnp.float32),
        mesh=plsc.VectorSubcoreMesh(core_axis_name="c", subcore_axis_name="s",
                                    num_cores=_NC, num_subcores=_NS),
        scratch_types=(
            [pltpu.VMEM((_SECT, _CHUNK), jnp.int32),
             pltpu.VMEM((_SECT, _CHUNK), jnp.int32),
             pltpu.VMEM((16,), jnp.int32)]
            + [pltpu.VMEM((_CHUNK, 2, _D // 2), jnp.float32)
               for _ in range(_NBUF)]
            + [pltpu.VMEM_SHARED((_SLAB, 2, _D // 2), jnp.float32)]
            + [pltpu.SemaphoreType.DMA for _ in range(_NBUF)]
        ),
    )


def _dot(a, b):
    return jnp.dot(a, b, preferred_element_type=jnp.float32)


def _store_tr(tr, tr_ref):
    for t in range(_NT):
        tr_ref[t] = tr[:, t * _D:(t + 1) * _D]


def _tr_body(h_ref, wtr_ref, tr_ref):
    _store_tr(_dot(h_ref[...], wtr_ref[...]), tr_ref)


_tr_out_spec = pl.BlockSpec((_NT, _BLK, _D), lambda i: (0, i, 0))
_tr_out_shape = jax.ShapeDtypeStruct((_NT, _N, _D), jnp.float32)

_tc_transform = pl.pallas_call(
    _tr_body,
    grid=(_NBLK,),
    in_specs=[
        pl.BlockSpec((_BLK, _D), lambda i: (i, 0)),
        pl.BlockSpec((_D, 3 * _D), lambda i: (0, 0)),
    ],
    out_specs=_tr_out_spec,
    out_shape=_tr_out_shape,
)


def _gru(agg_ref, h_ref, wih_ref, whh_ref, bih_ref, bhh_ref):
    gi = _dot(agg_ref[...], wih_ref[...]) + bih_ref[...]
    h = h_ref[...]
    gh = _dot(h, whh_ref[...]) + bhh_ref[...]
    r = jax.nn.sigmoid(gi[:, 0:_D] + gh[:, 0:_D])
    z = jax.nn.sigmoid(gi[:, _D:2 * _D] + gh[:, _D:2 * _D])
    n = jnp.tanh(gi[:, 2 * _D:] + r * gh[:, 2 * _D:])
    return (1.0 - z) * n + z * h


def _step_body(agg_ref, h_ref, wih_ref, whh_ref, bih_ref, bhh_ref,
               wtr_ref, h_out_ref, tr_ref):
    hn = _gru(agg_ref, h_ref, wih_ref, whh_ref, bih_ref, bhh_ref)
    h_out_ref[...] = hn
    _store_tr(_dot(hn, wtr_ref[...]), tr_ref)


_gru_in_specs = [
    pl.BlockSpec((_BLK, _D), lambda i: (i, 0)),
    pl.BlockSpec((_BLK, _D), lambda i: (i, 0)),
    pl.BlockSpec((_D, 3 * _D), lambda i: (0, 0)),
    pl.BlockSpec((_D, 3 * _D), lambda i: (0, 0)),
    pl.BlockSpec((1, 3 * _D), lambda i: (0, 0)),
    pl.BlockSpec((1, 3 * _D), lambda i: (0, 0)),
]

_tc_step = pl.pallas_call(
    _step_body,
    grid=(_NBLK,),
    in_specs=_gru_in_specs + [pl.BlockSpec((_D, 3 * _D), lambda i: (0, 0))],
    out_specs=[pl.BlockSpec((_BLK, _D), lambda i: (i, 0)), _tr_out_spec],
    out_shape=[jax.ShapeDtypeStruct((_N, _D), jnp.float32), _tr_out_shape],
)


def _final_body(agg_ref, h_ref, wih_ref, whh_ref, bih_ref, bhh_ref,
                wgp_ref, bgp_ref, ids_ref, out_ref):
    i = pl.program_id(0)
    hn = _gru(agg_ref, h_ref, wih_ref, whh_ref, bih_ref, bhh_ref)
    gp = _dot(hn, wgp_ref[...]) + bgp_ref[...]
    gated = jax.nn.sigmoid(gp[:, 0:1]) * gp[:, 1:2]
    ids = ids_ref[0]
    gids = lax.broadcasted_iota(jnp.int32, (_BLK, _G), 1)
    m = jnp.where(ids == gids, gated, 0.0)
    part = jnp.sum(m, axis=0, keepdims=True)

    @pl.when(i == 0)
    def _():
        out_ref[...] = jnp.zeros_like(out_ref)

    out_ref[...] += part


_tc_final = pl.pallas_call(
    _final_body,
    grid=(_NBLK,),
    in_specs=_gru_in_specs + [
        pl.BlockSpec((_D, 2), lambda i: (0, 0)),
        pl.BlockSpec((1, 2), lambda i: (0, 0)),
        pl.BlockSpec((1, _BLK, 1), lambda i: (i, 0, 0)),
    ],
    out_specs=pl.BlockSpec((1, _G), lambda i: (0, 0)),
    out_shape=jax.ShapeDtypeStruct((1, _G), jnp.float32),
)


def kernel(node_features, edge_index, edge_type, node_to_graph_id,
           A, W_ih, W_hh, b_ih, b_hh, Wp, bp, Wg, bg):
    src = edge_index[0]
    dst = edge_index[1]

    g = edge_type * _N + src
    lo = dst < _HALFN
    c0 = jnp.cumsum(lo.astype(jnp.int32))
    c1 = jnp.cumsum(1 - lo.astype(jnp.int32))
    n0 = c0[-1]
    n1 = _E - n0
    cdiv = lambda a, b: (a + b - 1) // b
    rnd = lambda x: cdiv(x, _CPS) * _CPS
    chunks0 = rnd(cdiv(n0, _CHUNK))
    chunks1 = rnd(cdiv(n1, _CHUNK))
    base1 = chunks0
    pos = jnp.where(lo, c0 - 1, base1 * _CHUNK + c1 - 1)
    ldst = jnp.where(lo, dst, dst - _HALFN)
    gidx = jnp.zeros((_CAPR * _CHUNK,), jnp.int32).at[pos].set(g)
    didx = jnp.full((_CAPR * _CHUNK,), _DUMMY, jnp.int32).at[pos].set(ldst)
    gidx = gidx.reshape(_CAPR, _CHUNK)
    didx = didx.reshape(_CAPR, _CHUNK)
    k0 = chunks0 // _NS
    k1 = chunks1 // _NS
    pad = jnp.zeros((13,), jnp.int32)
    scal = jnp.stack([
        jnp.concatenate([jnp.stack([jnp.int32(0), k0 // 8, k0 // _SECT]),
                         pad]),
        jnp.concatenate([jnp.stack([base1 // 8, k1 // 8, k1 // _SECT]),
                         pad]),
    ]).astype(jnp.int32)
    zeros = jnp.zeros((_ROWS_PT, 2, _D // 2), jnp.float32)

    w_tr = jnp.concatenate([A[0].T, A[1].T, A[2].T], axis=1)
    w_iht = W_ih.T
    w_hht = W_hh.T
    bih2 = b_ih.reshape(1, 3 * _D)
    bhh2 = b_hh.reshape(1, 3 * _D)
    wgp = jnp.concatenate([Wg, Wp], axis=1)
    bgp = jnp.concatenate([bg, bp]).reshape(1, 2)
    ids3 = node_to_graph_id.reshape(_NBLK, _BLK, 1)

    h = node_features
    tr = _tc_transform(h, w_tr)
    for t in range(_T):
        table = tr.reshape(_NT * _N, 2, _D // 2)
        out2 = _get_sc_agg()(table, gidx, didx, scal, zeros)
        out2 = out2.reshape(_NC, _SLAB, _D)
        agg = jnp.concatenate([out2[0, :_HALFN], out2[1, :_N - _HALFN]],
                              axis=0)
        if t < _T - 1:
            h, tr = _tc_step(agg, h, w_iht, w_hht, bih2, bhh2, w_tr)
        else:
            out = _tc_final(agg, h, w_iht, w_hht, bih2, bhh2,
                            wgp, bgp, ids3)
    return out.reshape(_G, 1)


# speedup vs baseline: 3.9444x; 3.9444x over previous
import functools

import jax
import jax.numpy as jnp
from jax import lax
from jax.experimental import pallas as pl
from jax.experimental.pallas import tpu as pltpu
from jax.experimental.pallas import tpu_sc as plsc

_N = 10000
_E = 160000
_D = 256
_NT = 3
_T = 4
_G = 64

_NC = 2
_NS = 16
_H = 128

_SLAB = 10112
_ROWS_PT = _SLAB // _NS
_DUMMY = 10080
_EPT = 10240
_CHUNK = 64
_NBUF = 5
_SECT = 4 * _NBUF
_NSECT = _EPT // (_SECT * _CHUNK)

_BLK = 1000
_NBLK = _N // _BLK


def _sc_agg_body(tr_hbm, gidx_hbm, didx_hbm, zeros_hbm, out_hbm,
                 gidx_v, didx_v, buf0, buf1, buf2, buf3, buf4, slab,
                 sem0, sem1, sem2, sem3, sem4):
    c = lax.axis_index("c")
    s = lax.axis_index("s")

    pltpu.sync_copy(zeros_hbm, slab.at[pl.ds(s * _ROWS_PT, _ROWS_PT)])
    plsc.subcore_barrier()

    bufs = (buf0, buf1, buf2, buf3, buf4)
    sems = (sem0, sem1, sem2, sem3, sem4)

    def section_body(sec, carry):
        pltpu.sync_copy(gidx_hbm.at[c, s, sec], gidx_v)
        pltpu.sync_copy(didx_hbm.at[s, sec], didx_v)
        for b in range(_NBUF):
            pltpu.async_copy(tr_hbm.at[gidx_v.at[b]], bufs[b], sems[b])

        def ring_body(i, carry2):
            for b in range(_NBUF):
                j = i * _NBUF + b
                pltpu.make_async_copy(tr_hbm.at[gidx_v.at[j]],
                                      bufs[b], sems[b]).wait()
                pltpu.sync_copy(bufs[b], slab.at[didx_v.at[j]], add=True)

                @pl.when(j + _NBUF < _SECT)
                def _():
                    pltpu.async_copy(tr_hbm.at[gidx_v.at[j + _NBUF]],
                                     bufs[b], sems[b])
            return carry2

        lax.fori_loop(0, _SECT // _NBUF, ring_body, 0)
        return carry

    lax.fori_loop(0, _NSECT, section_body, 0)

    plsc.subcore_barrier()
    pltpu.sync_copy(slab.at[pl.ds(s * _ROWS_PT, _ROWS_PT)],
                    out_hbm.at[c, pl.ds(s * _ROWS_PT, _ROWS_PT)])


@functools.cache
def _get_sc_agg():
    return pl.kernel(
        _sc_agg_body,
        out_type=jax.ShapeDtypeStruct((_NC, _SLAB, _H), jnp.float32),
        mesh=plsc.VectorSubcoreMesh(core_axis_name="c", subcore_axis_name="s",
                                    num_cores=_NC, num_subcores=_NS),
        scratch_types=(
            [pltpu.VMEM((_SECT, _CHUNK), jnp.int32),
             pltpu.VMEM((_SECT, _CHUNK), jnp.int32)]
            + [pltpu.VMEM((_CHUNK, _H), jnp.float32)
               for _ in range(_NBUF)]
            + [pltpu.VMEM_SHARED((_SLAB, _H), jnp.float32)]
            + [pltpu.SemaphoreType.DMA for _ in range(_NBUF)]
        ),
    )


def _dot(a, b):
    return jnp.dot(a, b, preferred_element_type=jnp.float32)


def _store_tr(tr, tr_ref):
    for t in range(_NT):
        tr_ref[t] = tr[:, t * _D:(t + 1) * _D]


def _tr_body(h_ref, wtr_ref, tr_ref):
    _store_tr(_dot(h_ref[...], wtr_ref[...]), tr_ref)


_tr_out_spec = pl.BlockSpec((_NT, _BLK, _D), lambda i: (0, i, 0))
_tr_out_shape = jax.ShapeDtypeStruct((_NT, _N, _D), jnp.float32)

_tc_transform = pl.pallas_call(
    _tr_body,
    grid=(_NBLK,),
    in_specs=[
        pl.BlockSpec((_BLK, _D), lambda i: (i, 0)),
        pl.BlockSpec((_D, 3 * _D), lambda i: (0, 0)),
    ],
    out_specs=_tr_out_spec,
    out_shape=_tr_out_shape,
)


def _gru(agg_ref, h_ref, wih_ref, whh_ref, bih_ref, bhh_ref):
    gi = (_dot(agg_ref[0], wih_ref[0:_H, :])
          + _dot(agg_ref[1], wih_ref[_H:_D, :]) + bih_ref[...])
    h = h_ref[...]
    gh = _dot(h, whh_ref[...]) + bhh_ref[...]
    r = jax.nn.sigmoid(gi[:, 0:_D] + gh[:, 0:_D])
    z = jax.nn.sigmoid(gi[:, _D:2 * _D] + gh[:, _D:2 * _D])
    n = jnp.tanh(gi[:, 2 * _D:] + r * gh[:, 2 * _D:])
    return (1.0 - z) * n + z * h


def _step_body(agg_ref, h_ref, wih_ref, whh_ref, bih_ref, bhh_ref,
               wtr_ref, h_out_ref, tr_ref):
    hn = _gru(agg_ref, h_ref, wih_ref, whh_ref, bih_ref, bhh_ref)
    h_out_ref[...] = hn
    _store_tr(_dot(hn, wtr_ref[...]), tr_ref)


_gru_in_specs = [
    pl.BlockSpec((_NC, _BLK, _H), lambda i: (0, i, 0)),
    pl.BlockSpec((_BLK, _D), lambda i: (i, 0)),
    pl.BlockSpec((_D, 3 * _D), lambda i: (0, 0)),
    pl.BlockSpec((_D, 3 * _D), lambda i: (0, 0)),
    pl.BlockSpec((1, 3 * _D), lambda i: (0, 0)),
    pl.BlockSpec((1, 3 * _D), lambda i: (0, 0)),
]

_tc_step = pl.pallas_call(
    _step_body,
    grid=(_NBLK,),
    in_specs=_gru_in_specs + [pl.BlockSpec((_D, 3 * _D), lambda i: (0, 0))],
    out_specs=[pl.BlockSpec((_BLK, _D), lambda i: (i, 0)), _tr_out_spec],
    out_shape=[jax.ShapeDtypeStruct((_N, _D), jnp.float32), _tr_out_shape],
)


def _final_body(agg_ref, h_ref, wih_ref, whh_ref, bih_ref, bhh_ref,
                wgp_ref, bgp_ref, ids_ref, out_ref):
    i = pl.program_id(0)
    hn = _gru(agg_ref, h_ref, wih_ref, whh_ref, bih_ref, bhh_ref)
    gp = _dot(hn, wgp_ref[...]) + bgp_ref[...]
    gated = jax.nn.sigmoid(gp[:, 0:1]) * gp[:, 1:2]
    ids = ids_ref[0]
    gids = lax.broadcasted_iota(jnp.int32, (_BLK, _G), 1)
    m = jnp.where(ids == gids, gated, 0.0)
    part = jnp.sum(m, axis=0, keepdims=True)

    @pl.when(i == 0)
    def _():
        out_ref[...] = jnp.zeros_like(out_ref)

    out_ref[...] += part


_tc_final = pl.pallas_call(
    _final_body,
    grid=(_NBLK,),
    in_specs=_gru_in_specs + [
        pl.BlockSpec((_D, 2), lambda i: (0, 0)),
        pl.BlockSpec((1, 2), lambda i: (0, 0)),
        pl.BlockSpec((1, _BLK, 1), lambda i: (i, 0, 0)),
    ],
    out_specs=pl.BlockSpec((1, _G), lambda i: (0, 0)),
    out_shape=jax.ShapeDtypeStruct((1, _G), jnp.float32),
)


def _per_tile_pad(x, fill):
    ept = _E // _NS
    x = x.reshape(_NS, ept)
    x = jnp.pad(x, ((0, 0), (0, _EPT - ept)), constant_values=fill)
    return x.reshape(_NS, _NSECT, _SECT, _CHUNK)


def kernel(node_features, edge_index, edge_type, node_to_graph_id,
           A, W_ih, W_hh, b_ih, b_hh, Wp, bp, Wg, bg):
    src = edge_index[0]
    dst = edge_index[1]

    g2 = (edge_type * _N + src) * 2
    gidx = jnp.stack([g2, g2 + 1])
    ept = _E // _NS
    gidx = gidx.reshape(_NC, _NS, ept)
    gidx = jnp.pad(gidx, ((0, 0), (0, 0), (0, _EPT - ept)))
    gidx = gidx.reshape(_NC, _NS, _NSECT, _SECT, _CHUNK)
    didx = _per_tile_pad(dst, _DUMMY)
    zeros = jnp.zeros((_ROWS_PT, _H), jnp.float32)

    w_tr = jnp.concatenate([A[0].T, A[1].T, A[2].T], axis=1)
    w_iht = W_ih.T
    w_hht = W_hh.T
    bih2 = b_ih.reshape(1, 3 * _D)
    bhh2 = b_hh.reshape(1, 3 * _D)
    wgp = jnp.concatenate([Wg, Wp], axis=1)
    bgp = jnp.concatenate([bg, bp]).reshape(1, 2)
    ids3 = node_to_graph_id.reshape(_NBLK, _BLK, 1)

    h = node_features
    tr = _tc_transform(h, w_tr)
    for t in range(_T):
        table = tr.reshape(2 * _NT * _N, _H)
        agg = _get_sc_agg()(table, gidx, didx, zeros)
        if t < _T - 1:
            h, tr = _tc_step(agg, h, w_iht, w_hht, bih2, bhh2, w_tr)
        else:
            out = _tc_final(agg, h, w_iht, w_hht, bih2, bhh2,
                            wgp, bgp, ids3)
    return out.reshape(_G, 1)
